# baseline (speedup 1.0000x reference)
import jax
import jax.numpy as jnp
from jax import lax
from jax.experimental import pallas as pl
from jax.experimental.pallas import tpu as pltpu

_TRHS = (((1,), (1,)), ((), ()))


def _group_expand(sb, ng, n_in, g):
    sel = (lax.broadcasted_iota(jnp.int32, (ng, n_in), 1) // g
           == lax.broadcasted_iota(jnp.int32, (ng, n_in), 0))
    return jnp.dot(sb.astype(jnp.bfloat16), sel.astype(jnp.bfloat16),
                   preferred_element_type=jnp.float32)


def _moe_tile_body(te_ref, x_ref, rw_ref, w1_ref, s1_ref, b1_ref,
                   w2_ref, s2_ref, b2_ref, o_ref, w1f_ref, w2f_ref):
    two_i, h = w1_ref.shape[1], w1_ref.shape[2]
    ng1 = s1_ref.shape[2]
    g1 = h // ng1
    ii = w2_ref.shape[2]
    ng2 = s2_ref.shape[2]
    g2 = ii // ng2

    t = pl.program_id(0)
    te_now = te_ref[t]
    te_prev = te_ref[jnp.maximum(t - 1, 0)]

    @pl.when((t == 0) | (te_now != te_prev))
    def _dequant():
        s1f = _group_expand(s1_ref[te_now], ng1, h, g1)
        b1f = _group_expand(b1_ref[te_now], ng1, h, g1)
        q1 = w1_ref[te_now].astype(jnp.float32)
        w1f_ref[...] = (q1 * s1f + b1f).astype(jnp.bfloat16)
        s2f = _group_expand(s2_ref[te_now], ng2, ii, g2)
        b2f = _group_expand(b2_ref[te_now], ng2, ii, g2)
        q2 = w2_ref[te_now].astype(jnp.float32)
        w2f_ref[...] = (q2 * s2f + b2f).astype(jnp.bfloat16)

    x = x_ref[...]
    y = lax.dot_general(x, w1f_ref[...], _TRHS,
                        preferred_element_type=jnp.float32)
    half = two_i // 2
    gate = y[:, :half]
    up = y[:, half:]
    act = gate * jax.nn.sigmoid(gate) * up
    z = lax.dot_general(act.astype(jnp.bfloat16), w2f_ref[...], _TRHS,
                        preferred_element_type=jnp.float32)
    o_ref[...] = z * rw_ref[0, 0][:, None]


def kernel(x, expert_weights, expert_indices, top_k, w1, s1, b1, w2, s2, b2):
    p, h = x.shape
    e, two_i, _ = w1.shape
    i = w2.shape[2]
    tk = expert_indices.shape[-1]
    t = p * tk
    B = 128
    nt = (t + e * (B - 1)) // B

    flat = expert_indices.reshape(-1).astype(jnp.int32)
    order = jnp.argsort(flat).astype(jnp.int32)
    sorted_e = flat[order]
    counts = jnp.bincount(flat, length=e).astype(jnp.int32)
    tiles_per = (counts + B - 1) // B
    tile_start = jnp.cumsum(tiles_per) - tiles_per
    offs = jnp.cumsum(counts) - counts
    jpos = jnp.arange(t, dtype=jnp.int32)
    dest = tile_start[sorted_e] * B + (jpos - offs[sorted_e])
    src = jnp.zeros((nt * B,), jnp.int32).at[dest].set(order)
    rw_flat = expert_weights.reshape(-1).astype(jnp.float32)
    rw_pad = jnp.zeros((nt * B,), jnp.float32).at[dest].set(rw_flat[order])
    tile_expert = jnp.minimum(
        jnp.searchsorted(jnp.cumsum(tiles_per), jnp.arange(nt), side="right"),
        e - 1).astype(jnp.int32)

    x_pad = jnp.take(x.astype(jnp.bfloat16), src // tk, axis=0)
    rw3 = rw_pad.reshape(nt, 1, B)

    w1q = w1.astype(jnp.int8)
    w2q = w2.astype(jnp.int8)

    _whole3 = lambda sh: pl.BlockSpec(sh, lambda ti, te: (0, 0, 0))
    grid_spec = pltpu.PrefetchScalarGridSpec(
        num_scalar_prefetch=1,
        grid=(nt,),
        in_specs=[
            pl.BlockSpec((B, h), lambda ti, te: (ti, 0)),
            pl.BlockSpec((1, 1, B), lambda ti, te: (ti, 0, 0)),
            _whole3(w1q.shape),
            _whole3(s1.shape),
            _whole3(b1.shape),
            _whole3(w2q.shape),
            _whole3(s2.shape),
            _whole3(b2.shape),
        ],
        out_specs=pl.BlockSpec((B, h), lambda ti, te: (ti, 0)),
        scratch_shapes=[
            pltpu.VMEM((two_i, h), jnp.bfloat16),
            pltpu.VMEM((h, i), jnp.bfloat16),
        ],
    )
    z_pad = pl.pallas_call(
        _moe_tile_body,
        grid_spec=grid_spec,
        out_shape=jax.ShapeDtypeStruct((nt * B, h), jnp.float32),
    )(tile_expert, x_pad, rw3, w1q, s1, b1, w2q, s2, b2)

    dest_flat = jnp.zeros((t,), jnp.int32).at[order].set(dest)
    return jnp.take(z_pad, dest_flat, axis=0).reshape(p, tk, h).sum(axis=1)


# speedup vs baseline: 1.1562x; 1.1562x over previous
import jax
import jax.numpy as jnp
from jax import lax
from jax.experimental import pallas as pl
from jax.experimental.pallas import tpu as pltpu

_TRHS = (((1,), (1,)), ((), ()))


def _group_expand(sb, ng, n_in, g):
    sel = (lax.broadcasted_iota(jnp.int32, (ng, n_in), 1) // g
           == lax.broadcasted_iota(jnp.int32, (ng, n_in), 0))
    return jnp.dot(sb.astype(jnp.bfloat16), sel.astype(jnp.bfloat16),
                   preferred_element_type=jnp.float32)


def _moe_tile_body(te_ref, x_ref, rw_ref, w1_ref, s1_ref, b1_ref,
                   w2_ref, s2_ref, b2_ref, o_ref, w1f_ref, w2f_ref):
    two_i, h = w1_ref.shape[1], w1_ref.shape[2]
    ng1 = s1_ref.shape[2]
    g1 = h // ng1
    ii = w2_ref.shape[2]
    ng2 = s2_ref.shape[2]
    g2 = ii // ng2

    t = pl.program_id(0)
    te_now = te_ref[t]
    te_prev = te_ref[jnp.maximum(t - 1, 0)]

    @pl.when((t == 0) | (te_now != te_prev))
    def _dequant():
        s1f = _group_expand(s1_ref[te_now], ng1, h, g1)
        b1f = _group_expand(b1_ref[te_now], ng1, h, g1)
        q1 = w1_ref[te_now].astype(jnp.float32)
        w1f_ref[...] = (q1 * s1f + b1f).astype(jnp.bfloat16)
        s2f = _group_expand(s2_ref[te_now], ng2, ii, g2)
        b2f = _group_expand(b2_ref[te_now], ng2, ii, g2)
        q2 = w2_ref[te_now].astype(jnp.float32)
        w2f_ref[...] = (q2 * s2f + b2f).astype(jnp.bfloat16)

    x = x_ref[...]
    y = lax.dot_general(x, w1f_ref[...], _TRHS,
                        preferred_element_type=jnp.float32)
    half = two_i // 2
    gate = y[:, :half]
    up = y[:, half:]
    act = gate * jax.nn.sigmoid(gate) * up
    z = lax.dot_general(act.astype(jnp.bfloat16), w2f_ref[...], _TRHS,
                        preferred_element_type=jnp.float32)
    o_ref[...] = z * rw_ref[0, 0][:, None]


def kernel(x, expert_weights, expert_indices, top_k, w1, s1, b1, w2, s2, b2):
    p, h = x.shape
    e, two_i, _ = w1.shape
    i = w2.shape[2]
    tk = expert_indices.shape[-1]
    t = p * tk
    B = 128
    nt = (t + e * (B - 1)) // B

    flat = expert_indices.reshape(-1).astype(jnp.int32)
    onehot = (flat[:, None] == jnp.arange(e, dtype=jnp.int32)[None, :])
    csum = jnp.cumsum(onehot.astype(jnp.int32), axis=0)
    rank = jnp.sum(jnp.where(onehot, csum, 0), axis=1) - 1
    counts = csum[-1]
    tiles_per = (counts + B - 1) // B
    tile_start = jnp.cumsum(tiles_per) - tiles_per
    dest_flat = tile_start[flat] * B + rank
    src = jnp.full((nt * B,), -1, jnp.int32).at[dest_flat].set(
        jnp.arange(t, dtype=jnp.int32))
    valid = src >= 0
    srcc = jnp.where(valid, src, 0)
    rw_flat = expert_weights.reshape(-1).astype(jnp.float32)
    rw_pad = jnp.where(valid, rw_flat[srcc], 0.0)
    tile_expert = jnp.minimum(
        jnp.searchsorted(jnp.cumsum(tiles_per), jnp.arange(nt), side="right"),
        e - 1).astype(jnp.int32)

    x_pad = jnp.take(x.astype(jnp.bfloat16), srcc // tk, axis=0)
    rw3 = rw_pad.reshape(nt, 1, B)

    w1q = w1.astype(jnp.int8)
    w2q = w2.astype(jnp.int8)

    _whole3 = lambda sh: pl.BlockSpec(sh, lambda ti, te: (0, 0, 0))
    grid_spec = pltpu.PrefetchScalarGridSpec(
        num_scalar_prefetch=1,
        grid=(nt,),
        in_specs=[
            pl.BlockSpec((B, h), lambda ti, te: (ti, 0)),
            pl.BlockSpec((1, 1, B), lambda ti, te: (ti, 0, 0)),
            _whole3(w1q.shape),
            _whole3(s1.shape),
            _whole3(b1.shape),
            _whole3(w2q.shape),
            _whole3(s2.shape),
            _whole3(b2.shape),
        ],
        out_specs=pl.BlockSpec((B, h), lambda ti, te: (ti, 0)),
        scratch_shapes=[
            pltpu.VMEM((two_i, h), jnp.bfloat16),
            pltpu.VMEM((h, i), jnp.bfloat16),
        ],
    )
    z_pad = pl.pallas_call(
        _moe_tile_body,
        grid_spec=grid_spec,
        out_shape=jax.ShapeDtypeStruct((nt * B, h), jnp.float32),
    )(tile_expert, x_pad, rw3, w1q, s1, b1, w2q, s2, b2)

    return jnp.take(z_pad, dest_flat, axis=0).reshape(p, tk, h).sum(axis=1)


# speedup vs baseline: 1.4155x; 1.2242x over previous
import jax
import jax.numpy as jnp
from jax import lax
from jax.experimental import pallas as pl
from jax.experimental.pallas import tpu as pltpu

_TRHS = (((1,), (1,)), ((), ()))


def _group_expand(sb, ng, n_in, g):
    sel = (lax.broadcasted_iota(jnp.int32, (ng, n_in), 1) // g
           == lax.broadcasted_iota(jnp.int32, (ng, n_in), 0))
    return jnp.dot(sb.astype(jnp.bfloat16), sel.astype(jnp.bfloat16),
                   preferred_element_type=jnp.float32)


def _moe_tile_body(te_ref, x_ref, w1_ref, s1_ref, b1_ref,
                   w2_ref, s2_ref, b2_ref, o_ref, w1f_ref, w2f_ref):
    two_i, h = w1_ref.shape[1], w1_ref.shape[2]
    ng1 = s1_ref.shape[2]
    g1 = h // ng1
    ii = w2_ref.shape[2]
    ng2 = s2_ref.shape[2]
    g2 = ii // ng2

    t = pl.program_id(0)
    te_now = te_ref[t]
    te_prev = te_ref[jnp.maximum(t - 1, 0)]

    @pl.when((t == 0) | (te_now != te_prev))
    def _dequant():
        s1f = _group_expand(s1_ref[te_now], ng1, h, g1)
        b1f = _group_expand(b1_ref[te_now], ng1, h, g1)
        q1 = w1_ref[te_now].astype(jnp.float32)
        w1f_ref[...] = (q1 * s1f + b1f).astype(jnp.bfloat16)
        s2f = _group_expand(s2_ref[te_now], ng2, ii, g2)
        b2f = _group_expand(b2_ref[te_now], ng2, ii, g2)
        q2 = w2_ref[te_now].astype(jnp.float32)
        w2f_ref[...] = (q2 * s2f + b2f).astype(jnp.bfloat16)

    x = x_ref[...]
    y = lax.dot_general(x, w1f_ref[...], _TRHS,
                        preferred_element_type=jnp.float32)
    half = two_i // 2
    gate = y[:, :half]
    up = y[:, half:]
    act = gate * jax.nn.sigmoid(gate) * up
    z = lax.dot_general(act.astype(jnp.bfloat16), w2f_ref[...], _TRHS,
                        preferred_element_type=jnp.float32)
    o_ref[...] = z.astype(jnp.bfloat16)


def kernel(x, expert_weights, expert_indices, top_k, w1, s1, b1, w2, s2, b2):
    p, h = x.shape
    e, two_i, _ = w1.shape
    i = w2.shape[2]
    tk = expert_indices.shape[-1]
    t = p * tk
    B = 128
    nt = (t + e * (B - 1)) // B

    flat = expert_indices.reshape(-1).astype(jnp.int32)
    onehot = (flat[:, None] == jnp.arange(e, dtype=jnp.int32)[None, :])
    csum = jnp.cumsum(onehot.astype(jnp.int32), axis=0)
    rank = jnp.sum(jnp.where(onehot, csum, 0), axis=1) - 1
    counts = csum[-1]
    tiles_per = (counts + B - 1) // B
    tile_start = jnp.cumsum(tiles_per) - tiles_per
    dest_flat = tile_start[flat] * B + rank
    src = jnp.zeros((nt * B,), jnp.int32).at[dest_flat].set(
        jnp.arange(t, dtype=jnp.int32))
    tile_expert = jnp.minimum(
        jnp.searchsorted(jnp.cumsum(tiles_per), jnp.arange(nt), side="right"),
        e - 1).astype(jnp.int32)

    x_pad = jnp.take(x.astype(jnp.bfloat16), src // tk, axis=0)

    w1q = w1.astype(jnp.int8)
    w2q = w2.astype(jnp.int8)

    _whole3 = lambda sh: pl.BlockSpec(sh, lambda ti, te: (0, 0, 0))
    grid_spec = pltpu.PrefetchScalarGridSpec(
        num_scalar_prefetch=1,
        grid=(nt,),
        in_specs=[
            pl.BlockSpec((B, h), lambda ti, te: (ti, 0)),
            _whole3(w1q.shape),
            _whole3(s1.shape),
            _whole3(b1.shape),
            _whole3(w2q.shape),
            _whole3(s2.shape),
            _whole3(b2.shape),
        ],
        out_specs=pl.BlockSpec((B, h), lambda ti, te: (ti, 0)),
        scratch_shapes=[
            pltpu.VMEM((two_i, h), jnp.bfloat16),
            pltpu.VMEM((h, i), jnp.bfloat16),
        ],
    )
    z_pad = pl.pallas_call(
        _moe_tile_body,
        grid_spec=grid_spec,
        out_shape=jax.ShapeDtypeStruct((nt * B, h), jnp.bfloat16),
    )(tile_expert, x_pad, w1q, s1, b1, w2q, s2, b2)

    dpt = dest_flat.reshape(p, tk)
    ew = expert_weights.astype(jnp.float32)
    out = jnp.take(z_pad, dpt[:, 0], axis=0).astype(jnp.float32) * ew[:, 0:1]
    for k in range(1, tk):
        out = out + jnp.take(z_pad, dpt[:, k], axis=0).astype(jnp.float32) * ew[:, k:k + 1]
    return out


# speedup vs baseline: 1.5874x; 1.1215x over previous
import jax
import jax.numpy as jnp
from jax import lax
from jax.experimental import pallas as pl
from jax.experimental.pallas import tpu as pltpu

_TRHS = (((1,), (1,)), ((), ()))


def _group_expand(sb, ng, n_in, g):
    sel = (lax.broadcasted_iota(jnp.int32, (ng, n_in), 1) // g
           == lax.broadcasted_iota(jnp.int32, (ng, n_in), 0))
    return jnp.dot(sb.astype(jnp.bfloat16), sel.astype(jnp.bfloat16),
                   preferred_element_type=jnp.float32)


def _moe_tile_body(te_ref, x_ref, w1_ref, s1_ref, b1_ref,
                   w2_ref, s2_ref, b2_ref, o_ref, w1f_ref, w2f_ref):
    two_i, h = w1_ref.shape[1], w1_ref.shape[2]
    ng1 = s1_ref.shape[2]
    g1 = h // ng1
    ii = w2_ref.shape[2]
    ng2 = s2_ref.shape[2]
    g2 = ii // ng2

    t = pl.program_id(0)
    te_now = te_ref[t]
    te_prev = te_ref[jnp.maximum(t - 1, 0)]

    @pl.when((t == 0) | (te_now != te_prev))
    def _dequant():
        s1f = _group_expand(s1_ref[te_now], ng1, h, g1)
        b1f = _group_expand(b1_ref[te_now], ng1, h, g1)
        q1 = w1_ref[te_now].astype(jnp.float32)
        w1f_ref[...] = (q1 * s1f + b1f).astype(jnp.bfloat16)
        s2f = _group_expand(s2_ref[te_now], ng2, ii, g2)
        b2f = _group_expand(b2_ref[te_now], ng2, ii, g2)
        q2 = w2_ref[te_now].astype(jnp.float32)
        w2f_ref[...] = (q2 * s2f + b2f).astype(jnp.bfloat16)

    x = x_ref[...]
    y = lax.dot_general(x, w1f_ref[...], _TRHS,
                        preferred_element_type=jnp.float32)
    half = two_i // 2
    gate = y[:, :half]
    up = y[:, half:]
    act = gate * jax.nn.sigmoid(gate) * up
    z = lax.dot_general(act.astype(jnp.bfloat16), w2f_ref[...], _TRHS,
                        preferred_element_type=jnp.float32)
    o_ref[...] = z.astype(jnp.bfloat16)


def kernel(x, expert_weights, expert_indices, top_k, w1, s1, b1, w2, s2, b2):
    p, h = x.shape
    e, two_i, _ = w1.shape
    i = w2.shape[2]
    tk = expert_indices.shape[-1]
    t = p * tk
    B = 256
    nt = (t + e * (B - 1)) // B

    flat = expert_indices.reshape(-1).astype(jnp.int32)
    onehot = (flat[:, None] == jnp.arange(e, dtype=jnp.int32)[None, :])
    csum = jnp.cumsum(onehot.astype(jnp.int32), axis=0)
    rank = jnp.sum(jnp.where(onehot, csum, 0), axis=1) - 1
    counts = csum[-1]
    tiles_per = (counts + B - 1) // B
    tile_start = jnp.cumsum(tiles_per) - tiles_per
    dest_flat = tile_start[flat] * B + rank
    src = jnp.zeros((nt * B,), jnp.int32).at[dest_flat].set(
        jnp.arange(t, dtype=jnp.int32))
    tile_expert = jnp.minimum(
        jnp.searchsorted(jnp.cumsum(tiles_per), jnp.arange(nt), side="right"),
        e - 1).astype(jnp.int32)

    x_pad = jnp.take(x.astype(jnp.bfloat16), src // tk, axis=0)

    w1q = w1.astype(jnp.int8)
    w2q = w2.astype(jnp.int8)

    _whole3 = lambda sh: pl.BlockSpec(sh, lambda ti, te: (0, 0, 0))
    grid_spec = pltpu.PrefetchScalarGridSpec(
        num_scalar_prefetch=1,
        grid=(nt,),
        in_specs=[
            pl.BlockSpec((B, h), lambda ti, te: (ti, 0)),
            _whole3(w1q.shape),
            _whole3(s1.shape),
            _whole3(b1.shape),
            _whole3(w2q.shape),
            _whole3(s2.shape),
            _whole3(b2.shape),
        ],
        out_specs=pl.BlockSpec((B, h), lambda ti, te: (ti, 0)),
        scratch_shapes=[
            pltpu.VMEM((two_i, h), jnp.bfloat16),
            pltpu.VMEM((h, i), jnp.bfloat16),
        ],
    )
    z_pad = pl.pallas_call(
        _moe_tile_body,
        grid_spec=grid_spec,
        out_shape=jax.ShapeDtypeStruct((nt * B, h), jnp.bfloat16),
    )(tile_expert, x_pad, w1q, s1, b1, w2q, s2, b2)

    dpt = dest_flat.reshape(p, tk)
    ew = expert_weights.astype(jnp.float32)
    out = jnp.take(z_pad, dpt[:, 0], axis=0).astype(jnp.float32) * ew[:, 0:1]
    for k in range(1, tk):
        out = out + jnp.take(z_pad, dpt[:, k], axis=0).astype(jnp.float32) * ew[:, k:k + 1]
    return out
